# baseline (speedup 1.0000x reference)
import jax
import jax.numpy as jnp
from jax import lax
from jax.experimental import pallas as pl
from jax.experimental.pallas import tpu as pltpu
from jax.experimental.pallas import tpu_sc as plsc

N_ENTITIES = 100000
DIM = 128
LANES = 16
NC = 2
NS = 16
NW = NC * NS

CHUNK = 400


def _permute(x, idx):
    dnums = lax.GatherDimensionNumbers(
        offset_dims=(), collapsed_slice_dims=(0,), start_index_map=(0,))
    return lax.gather(
        x, idx[:, None], dimension_numbers=dnums, slice_sizes=(1,),
        mode=lax.GatherScatterMode.PROMISE_IN_BOUNDS)


def _rsqrt(tv):
    iv = plsc.bitcast(tv, jnp.int32)
    iv = jnp.int32(0x5F3759DF) - (iv >> 1)
    y = plsc.bitcast(iv, jnp.float32)
    y = y * (1.5 - (tv * 0.5) * y * y)
    y = y * (1.5 - (tv * 0.5) * y * y)
    return y


def _normalize_chunk(rows_v):
    lane = lax.iota(jnp.int32, LANES)

    last = jnp.full((LANES,), LANES - 1, jnp.int32)

    @plsc.parallel_loop(0, CHUNK, 1, unroll=1)
    def row_body(r):
        vs = [rows_v[r, pl.ds(j * LANES, LANES)] for j in range(DIM // LANES)]
        acc = vs[0] * vs[0]
        for v in vs[1:]:
            acc = acc + v * v
        total = _permute(plsc.cumsum(acc), last)
        total = jnp.maximum(total, jnp.float32(1e-24))
        scale = _rsqrt(total)
        for j, v in enumerate(vs):
            rows_v[r, pl.ds(j * LANES, LANES)] = v * scale


def _body(ids_hbm, table_hbm, out_hbm,
          idx_a, idx_b, rows_a, rows_b,
          sem_ga, sem_gb, sem_sa, sem_sb):
    wid = lax.axis_index("s") * NC + lax.axis_index("c")
    b_per_w = ids_hbm.shape[0] // NW
    n_chunks = b_per_w // CHUNK
    base_w = wid * b_per_w

    idx = [idx_a, idx_b]
    rows = [rows_a, rows_b]
    sem_g = [sem_ga, sem_gb]
    sem_s = [sem_sa, sem_sb]

    pltpu.sync_copy(ids_hbm.at[pl.ds(base_w, CHUNK)], idx[0])
    pltpu.async_copy(table_hbm.at[idx[0]], rows[0], sem_g[0])

    def pair_body(cp, carry):
        for cur in (0, 1):
            nxt = 1 - cur
            ci = cp * 2 + cur
            base = base_w + ci * CHUNK

            @pl.when(ci > 0)
            def _():
                pltpu.make_async_copy(
                    rows[nxt], out_hbm.at[pl.ds(base - CHUNK, CHUNK)],
                    sem_s[nxt]).wait()

            @pl.when(ci + 1 < n_chunks)
            def _():
                pltpu.sync_copy(ids_hbm.at[pl.ds(base + CHUNK, CHUNK)],
                                idx[nxt])
                pltpu.async_copy(table_hbm.at[idx[nxt]], rows[nxt],
                                 sem_g[nxt])

            pltpu.make_async_copy(table_hbm.at[idx[cur]], rows[cur],
                                  sem_g[cur]).wait()
            _normalize_chunk(rows[cur])
            pltpu.async_copy(rows[cur], out_hbm.at[pl.ds(base, CHUNK)],
                             sem_s[cur])
        return carry

    lax.fori_loop(0, n_chunks // 2, pair_body, 0, unroll=False)

    last = n_chunks - 1
    pltpu.make_async_copy(
        rows[1], out_hbm.at[pl.ds(base_w + last * CHUNK, CHUNK)],
        sem_s[1]).wait()


@jax.jit
def _lookup_normalize(ids_flat, table):
    b = ids_flat.shape[0]
    mesh = plsc.VectorSubcoreMesh(core_axis_name="c", subcore_axis_name="s")
    return pl.kernel(
        _body,
        out_type=jax.ShapeDtypeStruct((b, DIM), jnp.float32),
        mesh=mesh,
        compiler_params=pltpu.CompilerParams(needs_layout_passes=False),
        scratch_types=[
            pltpu.VMEM((CHUNK,), jnp.int32),
            pltpu.VMEM((CHUNK,), jnp.int32),
            pltpu.VMEM((CHUNK, DIM), jnp.float32),
            pltpu.VMEM((CHUNK, DIM), jnp.float32),
            pltpu.SemaphoreType.DMA,
            pltpu.SemaphoreType.DMA,
            pltpu.SemaphoreType.DMA,
            pltpu.SemaphoreType.DMA,
        ],
    )(ids_flat, table)


def kernel(ids, emb_weight):
    n_e, n_s = ids.shape
    ids_t = jnp.transpose(ids).reshape(-1).astype(jnp.int32)
    out = _lookup_normalize(ids_t, emb_weight)
    return out.reshape(n_s, n_e, DIM).transpose(1, 0, 2)


# speedup vs baseline: 1.0403x; 1.0403x over previous
import jax
import jax.numpy as jnp
from jax import lax
from jax.experimental import pallas as pl
from jax.experimental.pallas import tpu as pltpu
from jax.experimental.pallas import tpu_sc as plsc

N_ENTITIES = 100000
DIM = 128
LANES = 16
NC = 2
NS = 16
NW = NC * NS

CHUNK = 320


def _permute(x, idx):
    dnums = lax.GatherDimensionNumbers(
        offset_dims=(), collapsed_slice_dims=(0,), start_index_map=(0,))
    return lax.gather(
        x, idx[:, None], dimension_numbers=dnums, slice_sizes=(1,),
        mode=lax.GatherScatterMode.PROMISE_IN_BOUNDS)


def _rsqrt(tv):
    iv = plsc.bitcast(tv, jnp.int32)
    iv = jnp.int32(0x5F3759DF) - (iv >> 1)
    y = plsc.bitcast(iv, jnp.float32)
    y = y * (1.5 - (tv * 0.5) * y * y)
    y = y * (1.5 - (tv * 0.5) * y * y)
    return y


def _normalize_chunk(rows_v):
    lane = lax.iota(jnp.int32, LANES)

    last = jnp.full((LANES,), LANES - 1, jnp.int32)

    @plsc.parallel_loop(0, CHUNK, 1, unroll=2)
    def row_body(r):
        vs = [rows_v[r, pl.ds(j * LANES, LANES)] for j in range(DIM // LANES)]
        acc = vs[0] * vs[0]
        for v in vs[1:]:
            acc = acc + v * v
        total = _permute(plsc.cumsum(acc), last)
        total = jnp.maximum(total, jnp.float32(1e-24))
        scale = _rsqrt(total)
        for j, v in enumerate(vs):
            rows_v[r, pl.ds(j * LANES, LANES)] = v * scale


def _body(ids_hbm, table_hbm, out_hbm,
          idx_a, idx_b, rows_a, rows_b,
          sem_ga, sem_gb, sem_sa, sem_sb):
    wid = lax.axis_index("s") * NC + lax.axis_index("c")
    b_per_w = ids_hbm.shape[0] // NW
    n_chunks = b_per_w // CHUNK
    base_w = wid * b_per_w

    idx = [idx_a, idx_b]
    rows = [rows_a, rows_b]
    sem_g = [sem_ga, sem_gb]
    sem_s = [sem_sa, sem_sb]

    pltpu.sync_copy(ids_hbm.at[pl.ds(base_w, CHUNK)], idx[0])
    pltpu.async_copy(table_hbm.at[idx[0]], rows[0], sem_g[0])

    def pair_body(cp, carry):
        for cur in (0, 1):
            nxt = 1 - cur
            ci = cp * 2 + cur
            base = base_w + ci * CHUNK

            @pl.when(ci > 0)
            def _():
                pltpu.make_async_copy(
                    rows[nxt], out_hbm.at[pl.ds(base - CHUNK, CHUNK)],
                    sem_s[nxt]).wait()

            @pl.when(ci + 1 < n_chunks)
            def _():
                pltpu.sync_copy(ids_hbm.at[pl.ds(base + CHUNK, CHUNK)],
                                idx[nxt])
                pltpu.async_copy(table_hbm.at[idx[nxt]], rows[nxt],
                                 sem_g[nxt])

            pltpu.make_async_copy(table_hbm.at[idx[cur]], rows[cur],
                                  sem_g[cur]).wait()
            _normalize_chunk(rows[cur])
            pltpu.async_copy(rows[cur], out_hbm.at[pl.ds(base, CHUNK)],
                             sem_s[cur])
        return carry

    lax.fori_loop(0, n_chunks // 2, pair_body, 0, unroll=False)

    last = n_chunks - 1
    pltpu.make_async_copy(
        rows[1], out_hbm.at[pl.ds(base_w + last * CHUNK, CHUNK)],
        sem_s[1]).wait()


@jax.jit
def _lookup_normalize(ids_flat, table):
    b = ids_flat.shape[0]
    mesh = plsc.VectorSubcoreMesh(core_axis_name="c", subcore_axis_name="s")
    return pl.kernel(
        _body,
        out_type=jax.ShapeDtypeStruct((b, DIM), jnp.float32),
        mesh=mesh,
        compiler_params=pltpu.CompilerParams(needs_layout_passes=False),
        scratch_types=[
            pltpu.VMEM((CHUNK,), jnp.int32),
            pltpu.VMEM((CHUNK,), jnp.int32),
            pltpu.VMEM((CHUNK, DIM), jnp.float32),
            pltpu.VMEM((CHUNK, DIM), jnp.float32),
            pltpu.SemaphoreType.DMA,
            pltpu.SemaphoreType.DMA,
            pltpu.SemaphoreType.DMA,
            pltpu.SemaphoreType.DMA,
        ],
    )(ids_flat, table)


def kernel(ids, emb_weight):
    n_e, n_s = ids.shape
    ids_t = jnp.transpose(ids).reshape(-1).astype(jnp.int32)
    out = _lookup_normalize(ids_t, emb_weight)
    return out.reshape(n_s, n_e, DIM).transpose(1, 0, 2)


# speedup vs baseline: 1.1259x; 1.0822x over previous
import jax
import jax.numpy as jnp
from jax import lax
from jax.experimental import pallas as pl
from jax.experimental.pallas import tpu as pltpu
from jax.experimental.pallas import tpu_sc as plsc

N_ENTITIES = 100000
DIM = 128
LANES = 16
NC = 2
NS = 16
NW = NC * NS

CHUNK = 400


def _permute(x, idx):
    dnums = lax.GatherDimensionNumbers(
        offset_dims=(), collapsed_slice_dims=(0,), start_index_map=(0,))
    return lax.gather(
        x, idx[:, None], dimension_numbers=dnums, slice_sizes=(1,),
        mode=lax.GatherScatterMode.PROMISE_IN_BOUNDS)


def _rsqrt(tv):
    iv = plsc.bitcast(tv, jnp.int32)
    iv = jnp.int32(0x5F3759DF) - (iv >> 1)
    y = plsc.bitcast(iv, jnp.float32)
    y = y * (1.5 - (tv * 0.5) * y * y)
    y = y * (1.5 - (tv * 0.5) * y * y)
    return y


def _normalize_chunk(rows_v):
    last = jnp.full((LANES,), LANES - 1, jnp.int32)

    @plsc.parallel_loop(0, CHUNK, 1, unroll=2)
    def row_body(r):
        vs = [rows_v[r, pl.ds(j * LANES, LANES)] for j in range(DIM // LANES)]
        acc = vs[0] * vs[0]
        for v in vs[1:]:
            acc = acc + v * v
        total = _permute(plsc.cumsum(acc), last)
        total = jnp.maximum(total, jnp.float32(1e-24))
        scale = _rsqrt(total)
        for j, v in enumerate(vs):
            rows_v[r, pl.ds(j * LANES, LANES)] = v * scale


def _body(ids_hbm, table_hbm, out_hbm,
          idx_a, idx_b, rows_a, rows_b,
          sem_ga, sem_gb, sem_sa, sem_sb):
    wid = lax.axis_index("s") * NC + lax.axis_index("c")
    b_per_w = ids_hbm.shape[0] // NW
    n_chunks = b_per_w // CHUNK
    base_w = wid * b_per_w

    idx = [idx_a, idx_b]
    rows = [rows_a, rows_b]
    sem_g = [sem_ga, sem_gb]
    sem_s = [sem_sa, sem_sb]

    pltpu.sync_copy(ids_hbm.at[pl.ds(base_w, CHUNK)], idx[0])
    pltpu.async_copy(table_hbm.at[idx[0]], rows[0], sem_g[0])

    def pair_body(cp, carry):
        for cur in (0, 1):
            nxt = 1 - cur
            ci = cp * 2 + cur
            base = base_w + ci * CHUNK

            @pl.when(ci > 0)
            def _():
                pltpu.make_async_copy(
                    rows[nxt], out_hbm.at[pl.ds(base - CHUNK, CHUNK)],
                    sem_s[nxt]).wait()

            @pl.when(ci + 1 < n_chunks)
            def _():
                pltpu.sync_copy(ids_hbm.at[pl.ds(base + CHUNK, CHUNK)],
                                idx[nxt])
                pltpu.async_copy(table_hbm.at[idx[nxt]], rows[nxt],
                                 sem_g[nxt])

            pltpu.make_async_copy(table_hbm.at[idx[cur]], rows[cur],
                                  sem_g[cur]).wait()
            _normalize_chunk(rows[cur])
            pltpu.async_copy(rows[cur], out_hbm.at[pl.ds(base, CHUNK)],
                             sem_s[cur])
        return carry

    lax.fori_loop(0, n_chunks // 2, pair_body, 0, unroll=False)

    last = n_chunks - 1
    pltpu.make_async_copy(
        rows[1], out_hbm.at[pl.ds(base_w + last * CHUNK, CHUNK)],
        sem_s[1]).wait()


@jax.jit
def _lookup_normalize(ids_flat, table):
    b = ids_flat.shape[0]
    mesh = plsc.VectorSubcoreMesh(core_axis_name="c", subcore_axis_name="s")
    return pl.kernel(
        _body,
        out_type=jax.ShapeDtypeStruct((b, DIM), jnp.float32),
        mesh=mesh,
        compiler_params=pltpu.CompilerParams(needs_layout_passes=False),
        scratch_types=[
            pltpu.VMEM((CHUNK,), jnp.int32),
            pltpu.VMEM((CHUNK,), jnp.int32),
            pltpu.VMEM((CHUNK, DIM), jnp.float32),
            pltpu.VMEM((CHUNK, DIM), jnp.float32),
            pltpu.SemaphoreType.DMA,
            pltpu.SemaphoreType.DMA,
            pltpu.SemaphoreType.DMA,
            pltpu.SemaphoreType.DMA,
        ],
    )(ids_flat, table)


def kernel(ids, emb_weight):
    n_e, n_s = ids.shape
    ids_t = jnp.transpose(ids).reshape(-1).astype(jnp.int32)
    out = _lookup_normalize(ids_t, emb_weight)
    return out.reshape(n_s, n_e, DIM).transpose(1, 0, 2)
